# Initial kernel scaffold; baseline (speedup 1.0000x reference)
import jax
import jax.numpy as jnp
from jax.experimental import pallas as pl


def kernel(x, edge_index, W1, att_src1, att_dst1, b1, W2, att_src2, att_dst2, b2):
    raise NotImplementedError("write your pallas kernel here")


# speedup vs baseline: 58.2663x; 58.2663x over previous
import functools

import jax
import jax.numpy as jnp
from jax import lax
from jax.experimental import pallas as pl
from jax.experimental.pallas import tpu as pltpu
from jax.experimental.pallas import tpu_sc as plsc

N = 10000
D_IN = 128
H1, C1 = 8, 8
H2, C2 = 1, 7

NC, NS, LANES = 2, 16, 16
NW = NC * NS
K = 128
N_PAD = 10240
RPS = N_PAD // NS

W72 = H1 * C1 + H1
W8 = H1
W2_8 = C2 + 1

_MESH = plsc.VectorSubcoreMesh(
    core_axis_name="c", subcore_axis_name="s", num_cores=NC, num_subcores=NS)


def _sc_edge_pass_l1(table72, tableD, src, dst, zeros72, nchunks, e_total):
    T = nchunks * K

    @functools.partial(
        pl.kernel,
        out_type=jax.ShapeDtypeStruct((NC, N_PAD, W72), jnp.float32),
        mesh=_MESH,
        compiler_params=pltpu.CompilerParams(needs_layout_passes=False, use_tc_tiling_on_sc=False),
        scratch_types=[
            pltpu.VMEM_SHARED((N_PAD, W72), jnp.float32),
            pltpu.VMEM_SHARED((N, W8), jnp.float32),
            pltpu.VMEM((K,), jnp.int32),
            pltpu.VMEM((K,), jnp.int32),
            pltpu.VMEM((K, W72), jnp.float32),
            pltpu.VMEM((K, W8), jnp.float32),
            pltpu.VMEM((K, W72), jnp.float32),
        ],
    )
    def k(t72_hbm, tD_hbm, src_hbm, dst_hbm, z_hbm, part_hbm,
          accum, adshared, src_v, dst_v, rows_v, rowsd_v, out_v):
        cid = lax.axis_index("c")
        sid = lax.axis_index("s")
        wid = sid * NC + cid
        r0 = sid * RPS
        pltpu.sync_copy(z_hbm.at[pl.ds(r0, RPS)], accum.at[pl.ds(r0, RPS)])
        d0 = sid * (N // NS)
        pltpu.sync_copy(tD_hbm.at[pl.ds(d0, N // NS)], adshared.at[pl.ds(d0, N // NS)])
        plsc.subcore_barrier()

        iota = lax.iota(jnp.int32, LANES)
        base0 = wid * T

        def chunk(it, carry):
            base = base0 + it * K
            pltpu.sync_copy(src_hbm.at[pl.ds(base, K)], src_v)
            pltpu.sync_copy(dst_hbm.at[pl.ds(base, K)], dst_v)
            pltpu.sync_copy(t72_hbm.at[src_v], rows_v)
            pltpu.sync_copy(adshared.at[dst_v], rowsd_v)

            def group(g, carry2):
                rows = g * LANES + iota
                lane_ok = (base + g * LANES + iota) < e_total
                for hd in range(H1):
                    colw = jnp.full((LANES,), H1 * C1 + hd, jnp.int32)
                    a_s = plsc.load_gather(rows_v, [rows, colw])
                    a_d = plsc.load_gather(rowsd_v, [rows, jnp.full((LANES,), hd, jnp.int32)])
                    alpha = a_s + a_d
                    alpha = jnp.where(alpha >= 0, alpha, 0.2 * alpha)
                    w = jnp.where(lane_ok, jnp.exp(alpha), 0.0)
                    plsc.store_scatter(out_v, [rows, colw], w)
                    for c in range(C1):
                        col = jnp.full((LANES,), hd * C1 + c, jnp.int32)
                        hv = plsc.load_gather(rows_v, [rows, col])
                        plsc.store_scatter(out_v, [rows, col], w * hv)
                return carry2

            lax.fori_loop(0, K // LANES, group, 0)
            pltpu.sync_copy(out_v, accum.at[dst_v], add=True)
            return carry

        lax.fori_loop(0, nchunks, chunk, 0)
        plsc.subcore_barrier()
        pltpu.sync_copy(accum.at[pl.ds(r0, RPS)], part_hbm.at[cid, pl.ds(r0, RPS)])

    return k(table72, tableD, src, dst, zeros72)


def _sc_edge_pass_l2(table8, ad2, src, dst, zeros8, nchunks, e_total):
    T = nchunks * K

    @functools.partial(
        pl.kernel,
        out_type=jax.ShapeDtypeStruct((NC, N_PAD, W2_8), jnp.float32),
        mesh=_MESH,
        compiler_params=pltpu.CompilerParams(needs_layout_passes=False, use_tc_tiling_on_sc=False),
        scratch_types=[
            pltpu.VMEM_SHARED((N_PAD, W2_8), jnp.float32),
            pltpu.VMEM((N,), jnp.float32),
            pltpu.VMEM((K,), jnp.int32),
            pltpu.VMEM((K,), jnp.int32),
            pltpu.VMEM((K, W2_8), jnp.float32),
            pltpu.VMEM((K, W2_8), jnp.float32),
        ],
    )
    def k(t8_hbm, ad_hbm, src_hbm, dst_hbm, z_hbm, part_hbm,
          accum, adbuf, src_v, dst_v, rows_v, out_v):
        cid = lax.axis_index("c")
        sid = lax.axis_index("s")
        wid = sid * NC + cid
        r0 = sid * RPS
        pltpu.sync_copy(z_hbm.at[pl.ds(r0, RPS)], accum.at[pl.ds(r0, RPS)])
        pltpu.sync_copy(ad_hbm, adbuf)
        plsc.subcore_barrier()

        iota = lax.iota(jnp.int32, LANES)
        base0 = wid * T

        def chunk(it, carry):
            base = base0 + it * K
            pltpu.sync_copy(src_hbm.at[pl.ds(base, K)], src_v)
            pltpu.sync_copy(dst_hbm.at[pl.ds(base, K)], dst_v)
            pltpu.sync_copy(t8_hbm.at[src_v], rows_v)

            def group(g, carry2):
                rows = g * LANES + iota
                dvals = dst_v[pl.ds(g * LANES, LANES)]
                lane_ok = (base + g * LANES + iota) < e_total
                colw = jnp.full((LANES,), C2, jnp.int32)
                a_s = plsc.load_gather(rows_v, [rows, colw])
                a_d = plsc.load_gather(adbuf, [dvals])
                alpha = a_s + a_d
                alpha = jnp.where(alpha >= 0, alpha, 0.2 * alpha)
                w = jnp.where(lane_ok, jnp.exp(alpha), 0.0)
                plsc.store_scatter(out_v, [rows, colw], w)
                for c in range(C2):
                    col = jnp.full((LANES,), c, jnp.int32)
                    hv = plsc.load_gather(rows_v, [rows, col])
                    plsc.store_scatter(out_v, [rows, col], w * hv)
                return carry2

            lax.fori_loop(0, K // LANES, group, 0)
            pltpu.sync_copy(out_v, accum.at[dst_v], add=True)
            return carry

        lax.fori_loop(0, nchunks, chunk, 0)
        plsc.subcore_barrier()
        pltpu.sync_copy(accum.at[pl.ds(r0, RPS)], part_hbm.at[cid, pl.ds(r0, RPS)])

    return k(table8, ad2, src, dst, zeros8)


_BN = 1000


def _tc_tables1(x, wcat):
    def body(x_ref, w_ref, o72_ref, o8_ref):
        h = jnp.dot(x_ref[...], w_ref[...], preferred_element_type=jnp.float32)
        o72_ref[...] = h[:, :W72]
        o8_ref[...] = h[:, W72:]

    return pl.pallas_call(
        body,
        grid=(N // _BN,),
        in_specs=[pl.BlockSpec((_BN, D_IN), lambda i: (i, 0)),
                  pl.BlockSpec((D_IN, W72 + W8), lambda i: (0, 0))],
        out_specs=[pl.BlockSpec((_BN, W72), lambda i: (i, 0)),
                   pl.BlockSpec((_BN, W8), lambda i: (i, 0))],
        out_shape=[jax.ShapeDtypeStruct((N, W72), jnp.float32),
                   jax.ShapeDtypeStruct((N, W8), jnp.float32)],
    )(x, wcat)


def _tc_mid(part1, b1row, rrep, m8, adv):
    def body(p_ref, b_ref, r_ref, m_ref, a_ref, t8_ref, ad_ref):
        num = p_ref[0, :, :H1 * C1] + p_ref[1, :, :H1 * C1]
        den = p_ref[0, :, H1 * C1:] + p_ref[1, :, H1 * C1:]
        denr = jnp.dot(den, r_ref[...], preferred_element_type=jnp.float32)
        out1 = num / denr + b_ref[...]
        t8_ref[...] = jnp.dot(out1, m_ref[...], preferred_element_type=jnp.float32)
        ad_ref[...] = jnp.dot(out1, a_ref[...], preferred_element_type=jnp.float32)

    return pl.pallas_call(
        body,
        grid=(N // _BN,),
        in_specs=[pl.BlockSpec((NC, _BN, W72), lambda i: (0, i, 0)),
                  pl.BlockSpec((1, H1 * C1), lambda i: (0, 0)),
                  pl.BlockSpec((H1, H1 * C1), lambda i: (0, 0)),
                  pl.BlockSpec((H1 * C1, W2_8), lambda i: (0, 0)),
                  pl.BlockSpec((H1 * C1, 1), lambda i: (0, 0))],
        out_specs=[pl.BlockSpec((_BN, W2_8), lambda i: (i, 0)),
                   pl.BlockSpec((_BN, 1), lambda i: (i, 0))],
        out_shape=[jax.ShapeDtypeStruct((N, W2_8), jnp.float32),
                   jax.ShapeDtypeStruct((N, 1), jnp.float32)],
    )(part1, b1row, rrep, m8, adv)


def _tc_final(part2, b2row):
    def body(p_ref, b_ref, o_ref):
        num = p_ref[0, :, :C2] + p_ref[1, :, :C2]
        den = p_ref[0, :, C2:] + p_ref[1, :, C2:]
        o = num / den + b_ref[...]
        m = jnp.max(o, axis=1, keepdims=True)
        ex = jnp.exp(o - m)
        o_ref[...] = (o - m) - jnp.log(jnp.sum(ex, axis=1, keepdims=True))

    return pl.pallas_call(
        body,
        grid=(N // _BN,),
        in_specs=[pl.BlockSpec((NC, _BN, W2_8), lambda i: (0, i, 0)),
                  pl.BlockSpec((1, C2), lambda i: (0, 0))],
        out_specs=pl.BlockSpec((_BN, C2), lambda i: (i, 0)),
        out_shape=jax.ShapeDtypeStruct((N, C2), jnp.float32),
    )(part2, b2row)


def kernel(x, edge_index, W1, att_src1, att_dst1, b1, W2, att_src2, att_dst2, b2):
    W1r = W1.reshape(D_IN, H1, C1)
    wsrc1 = jnp.einsum("dhc,hc->dh", W1r, att_src1[0])
    wdst1 = jnp.einsum("dhc,hc->dh", W1r, att_dst1[0])
    wcat = jnp.concatenate([W1, wsrc1, wdst1], axis=1)

    m8 = jnp.concatenate([W2, (W2 @ att_src2[0, 0])[:, None]], axis=1)
    adv = (W2 @ att_dst2[0, 0])[:, None]
    rrep = jnp.repeat(jnp.eye(H1, dtype=jnp.float32), C1, axis=1)
    b1row = b1.reshape(1, H1 * C1)
    b2row = b2.reshape(1, C2)

    e_in = edge_index.shape[1]
    e_total = e_in + N
    nchunks = -(-e_total // (NW * K))
    e_pad = NW * nchunks * K
    loops = jnp.arange(N, dtype=jnp.int32)
    padz = jnp.zeros((e_pad - e_total,), jnp.int32)
    src = jnp.concatenate([edge_index[0].astype(jnp.int32), loops, padz])
    dst = jnp.concatenate([edge_index[1].astype(jnp.int32), loops, padz])

    zeros72 = jnp.zeros((N_PAD, W72), jnp.float32)
    zeros8 = jnp.zeros((N_PAD, W2_8), jnp.float32)

    table72, tableD = _tc_tables1(x, wcat)
    part1 = _sc_edge_pass_l1(table72, tableD, src, dst, zeros72, nchunks, e_total)
    table8, ad2 = _tc_mid(part1, b1row, rrep, m8, adv)
    part2 = _sc_edge_pass_l2(table8, ad2.reshape(N), src, dst, zeros8, nchunks, e_total)
    return _tc_final(part2, b2row)
